# baseline (speedup 1.0000x reference)
import functools

import jax
import jax.numpy as jnp
from jax import lax
from jax.experimental import pallas as pl
from jax.experimental.pallas import tpu as pltpu
from jax.experimental.pallas import tpu_sc as plsc

NUM_CORES = 2
NUM_SUBCORES = 16
NUM_WORKERS = NUM_CORES * NUM_SUBCORES
CHUNK = 128


@functools.partial(jax.jit, static_argnames=("nchunk", "d"))
def _gather_rows(idx3, weight, *, nchunk, d):
    bpw = nchunk * CHUNK
    mesh = plsc.VectorSubcoreMesh(core_axis_name="c", subcore_axis_name="s")

    @functools.partial(
        pl.kernel,
        out_type=jax.ShapeDtypeStruct((NUM_WORKERS * bpw, d), jnp.float32),
        mesh=mesh,
        scratch_types=[
            pltpu.VMEM((nchunk, CHUNK), jnp.int32),
            pltpu.VMEM((CHUNK, d), jnp.float32),
            pltpu.SemaphoreType.DMA,
        ],
        compiler_params=pltpu.CompilerParams(use_tc_tiling_on_sc=False),
    )
    def body(idx_hbm, tab_hbm, out_hbm, idx_v, rows_v, sem):
        cid = lax.axis_index("c")
        sid = lax.axis_index("s")
        wid = sid * NUM_CORES + cid
        base = wid * bpw
        pltpu.sync_copy(idx_hbm.at[wid], idx_v)

        def step(j, carry):
            pltpu.async_copy(tab_hbm.at[idx_v.at[j]], rows_v, sem).wait()
            pltpu.sync_copy(rows_v, out_hbm.at[pl.ds(base + j * CHUNK, CHUNK)])
            return carry

        lax.fori_loop(0, nchunk, step, 0)

    return body(idx3, weight)


def kernel(input, weight):
    batch, hist = input.shape
    d = weight.shape[1]
    total = batch * hist
    assert total % (NUM_WORKERS * CHUNK) == 0
    nchunk = total // (NUM_WORKERS * CHUNK)
    idx3 = input.reshape(NUM_WORKERS, nchunk, CHUNK)
    out = _gather_rows(idx3, weight, nchunk=nchunk, d=d)
    return out.reshape(batch, hist, d)


# speedup vs baseline: 1.0438x; 1.0438x over previous
import functools

import jax
import jax.numpy as jnp
from jax import lax
from jax.experimental import pallas as pl
from jax.experimental.pallas import tpu as pltpu
from jax.experimental.pallas import tpu_sc as plsc

NUM_CORES = 2
NUM_SUBCORES = 16
NUM_WORKERS = NUM_CORES * NUM_SUBCORES
CHUNK = 128


NBUF = 10
AHEAD = 7


@functools.partial(jax.jit, static_argnames=("nchunk", "d"))
def _gather_rows(idx3, weight, *, nchunk, d):
    bpw = nchunk * CHUNK
    ngroups = nchunk // NBUF
    assert nchunk % NBUF == 0
    mesh = plsc.VectorSubcoreMesh(core_axis_name="c", subcore_axis_name="s")

    @functools.partial(
        pl.kernel,
        out_type=jax.ShapeDtypeStruct((NUM_WORKERS * bpw, d), jnp.float32),
        mesh=mesh,
        scratch_types=[
            pltpu.VMEM((nchunk, CHUNK), jnp.int32),
            [pltpu.VMEM((CHUNK, d), jnp.float32) for _ in range(NBUF)],
            [pltpu.SemaphoreType.DMA for _ in range(NBUF)],
            [pltpu.SemaphoreType.DMA for _ in range(NBUF)],
        ],
        compiler_params=pltpu.CompilerParams(use_tc_tiling_on_sc=False),
    )
    def body(idx_hbm, tab_hbm, out_hbm, idx_v, bufs, gsems, ssems):
        cid = lax.axis_index("c")
        sid = lax.axis_index("s")
        wid = sid * NUM_CORES + cid
        base = wid * bpw
        pltpu.sync_copy(idx_hbm.at[wid], idx_v)

        def fire(j, b):
            pltpu.async_copy(tab_hbm.at[idx_v.at[j]], bufs[b], gsems[b])

        def gwait(j, b):
            pltpu.make_async_copy(tab_hbm.at[idx_v.at[j]], bufs[b], gsems[b]).wait()

        def store(j, b):
            pltpu.async_copy(bufs[b], out_hbm.at[pl.ds(base + j * CHUNK, CHUNK)], ssems[b])

        def swait(j, b):
            pltpu.make_async_copy(bufs[b], out_hbm.at[pl.ds(base + j * CHUNK, CHUNK)], ssems[b]).wait()

        for b in range(AHEAD):
            fire(b, b)

        def group(g, carry):
            j0 = g * NBUF
            for b in range(NBUF):
                j = j0 + b
                bf = (b + AHEAD) % NBUF
                jf = j + AHEAD
                @pl.when(jnp.logical_and(jf >= NBUF, jf < nchunk))
                def _():
                    swait(jf - NBUF, bf)

                @pl.when(jf < nchunk)
                def _():
                    fire(jf, bf)

                gwait(j, b)
                store(j, b)
            return carry

        lax.fori_loop(0, ngroups, group, 0)

        for b in range(NBUF):
            swait(nchunk - NBUF + b, b)

    return body(idx3, weight)


def kernel(input, weight):
    batch, hist = input.shape
    d = weight.shape[1]
    total = batch * hist
    assert total % (NUM_WORKERS * CHUNK) == 0
    nchunk = total // (NUM_WORKERS * CHUNK)
    idx3 = input.reshape(NUM_WORKERS, nchunk, CHUNK)
    out = _gather_rows(idx3, weight, nchunk=nchunk, d=d)
    return out.reshape(batch, hist, d)
